# baseline (speedup 1.0000x reference)
import jax
import jax.numpy as jnp
from jax import lax
from jax.experimental import pallas as pl
from jax.experimental.pallas import tpu as pltpu

_J, _K = 8, 7


def _copy_body(x_ref, o_ref, sem):
    bi = pl.program_id(0)
    copies = []
    for j in range(_J):
        c1 = pltpu.make_async_copy(x_ref.at[bi, :6], o_ref.at[bi, j, :6], sem)
        c2 = pltpu.make_async_copy(x_ref.at[bi, 6 + j], o_ref.at[bi, j, 6], sem)
        c1.start()
        c2.start()
        copies += [c1, c2]
    for c in copies:
        c.wait()


def kernel(x):
    b, s, n, m, d = x.shape
    return pl.pallas_call(
        _copy_body,
        grid=(b,),
        in_specs=[pl.BlockSpec(memory_space=pl.ANY)],
        out_specs=pl.BlockSpec(memory_space=pl.ANY),
        out_shape=jax.ShapeDtypeStruct((b, _J, _K, n, m, d), x.dtype),
        scratch_shapes=[pltpu.SemaphoreType.DMA],
    )(x)


# speedup vs baseline: 59.8316x; 59.8316x over previous
import jax
import jax.numpy as jnp
from jax import lax
from jax.experimental import pallas as pl
from jax.experimental.pallas import tpu as pltpu
from jax.experimental.pallas import tpu_sc as plsc

_J, _K = 8, 7


def _sc_body(x_hbm, o_hbm, buf, lsem, ssem):
    info = plsc.get_sparse_core_info()
    nc = info.num_cores
    wid = lax.axis_index("s") * nc + lax.axis_index("c")

    def row_copy(src, dsts):
        b, r = src
        ld = pltpu.make_async_copy(x_hbm.at[b, r], buf, lsem)
        ld.start()
        ld.wait()
        stores = [
            pltpu.make_async_copy(buf, o_hbm.at[bb, j, k], ssem)
            for (bb, j, k) in dsts
        ]
        for st in stores:
            st.start()
        for st in stores:
            st.wait()

    def bgroup(g):
        b, k = g // 6, g % 6
        row_copy((b, k), [(b, j, k) for j in range(_J)])

    def diag(t):
        b, j = t // _J, t % _J
        row_copy((b, 6 + j), [(b, j, 6)])

    @pl.when(wid < 16)
    def _():
        for i in range(2):
            bgroup(wid * 2 + i)

    @pl.when(wid >= 16)
    def _():
        bgroup(32 + (wid - 16))
        for i in range(4):
            diag((wid - 16) * 4 + i)


def kernel(x):
    b, s, n, m, d = x.shape
    xt = x.transpose(0, 1, 3, 2, 4)
    mesh = plsc.VectorSubcoreMesh(core_axis_name="c", subcore_axis_name="s")
    fn = pl.kernel(
        _sc_body,
        out_type=jax.ShapeDtypeStruct((b, _J, _K, m, n, d), x.dtype),
        mesh=mesh,
        scratch_types=[
            pltpu.VMEM((m, n, d), x.dtype),
            pltpu.SemaphoreType.DMA,
            pltpu.SemaphoreType.DMA,
        ],
    )
    return fn(xt).transpose(0, 1, 2, 4, 3, 5)
